# baseline (speedup 1.0000x reference)
import jax
import jax.numpy as jnp
from jax import lax
from jax.experimental import pallas as pl
from jax.experimental.pallas import tpu as pltpu
from jax.experimental.pallas import tpu_sc as plsc

B = 16
L = 2048
MARGIN = 0.1
LANES = 16
NCORE = 2
NSUB = 16
HALF = L // NCORE
NCHUNK = HALF // 128


def _sc_body(scores_hbm, wr_hbm, out_hbm, idx_v, vals_v, acc_v, zidx_v,
             hidx_v, hvals_v, shared, red_v, out_v, sem):
    cid = lax.axis_index("c")
    sid = lax.axis_index("s")
    row_chunk = sid * (L // 128)

    pltpu.sync_copy(wr_hbm.at[pl.ds(row_chunk + cid * NCHUNK, NCHUNK)], idx_v)
    copies = [
        pltpu.async_copy(scores_hbm.at[idx_v.at[i]], vals_v.at[i], sem)
        for i in range(NCHUNK)
    ]

    zidx_v[...] = jnp.zeros((LANES,), jnp.int32)

    @pl.when(sid == 0)
    def _():
        out_v[...] = jnp.zeros((LANES,), jnp.float32)
        pltpu.sync_copy(out_v, shared.at[0])

    @pl.when(cid == 1)
    def _():
        pltpu.sync_copy(wr_hbm.at[row_chunk, pl.ds(0, LANES)], hidx_v)
        pltpu.async_copy(scores_hbm.at[hidx_v], hvals_v, sem).wait()

    for c in copies:
        c.wait()

    top0 = vals_v[0, pl.ds(0, LANES)][0]
    toph = hvals_v[pl.ds(0, LANES)][0]
    top = jnp.where(cid == 0, top0, toph)

    acc = jnp.zeros((LANES,), jnp.float32)
    for i in range(NCHUNK):
        for k in range(128 // LANES):
            x = vals_v[i, pl.ds(k * LANES, LANES)]
            acc = acc + jnp.maximum(x - top + MARGIN, 0.0)
    acc_v[0, pl.ds(0, LANES)] = acc

    plsc.subcore_barrier()
    pltpu.sync_copy(acc_v, shared.at[zidx_v.at[pl.ds(0, 1)]], add=True)
    plsc.subcore_barrier()

    @pl.when(sid == 0)
    def _():
        pltpu.sync_copy(shared.at[0], red_v)
        pltpu.sync_copy(red_v, out_hbm.at[cid])


_mesh = plsc.VectorSubcoreMesh(core_axis_name="c", subcore_axis_name="s",
                               num_cores=NCORE, num_subcores=NSUB)

_sc_call = pl.kernel(
    _sc_body,
    out_type=jax.ShapeDtypeStruct((NCORE, LANES), jnp.float32),
    mesh=_mesh,
    scratch_types=[
        pltpu.VMEM((NCHUNK, 128), jnp.int32),
        pltpu.VMEM((NCHUNK, 128), jnp.float32),
        pltpu.VMEM((1, LANES), jnp.float32),
        pltpu.VMEM((LANES,), jnp.int32),
        pltpu.VMEM((LANES,), jnp.int32),
        pltpu.VMEM((LANES,), jnp.float32),
        pltpu.VMEM_SHARED((1, LANES), jnp.float32),
        pltpu.VMEM((LANES,), jnp.float32),
        pltpu.VMEM((LANES,), jnp.float32),
        pltpu.SemaphoreType.DMA,
    ],
    name="self_margin_loss_sc",
)


def _tc_body(part_ref, out_ref):
    s = jnp.sum(part_ref[...])
    out_ref[...] = jnp.full((1, 1), (s - B * MARGIN) * (1.0 / (L - 1)),
                            jnp.float32)


_tc_call = pl.pallas_call(
    _tc_body,
    out_shape=jax.ShapeDtypeStruct((1, 1), jnp.float32),
)


@jax.jit
def kernel(scores, nBestIndex, werRank):
    del nBestIndex
    parts = _sc_call(scores, werRank.reshape(-1, 128))
    return _tc_call(parts).reshape(1)


# speedup vs baseline: 1.0717x; 1.0717x over previous
import jax
import jax.numpy as jnp
from jax import lax
from jax.experimental import pallas as pl
from jax.experimental.pallas import tpu as pltpu
from jax.experimental.pallas import tpu_sc as plsc

B = 16
L = 2048
MARGIN = 0.1
LANES = 16
NSUB = 16
NCHUNK = L // 128


def _sc_body(scores_hbm, wr_hbm, out_hbm, idx_v, vals_v, acc_v, zidx_v,
             shared, red_v, out_v, sem):
    sid = lax.axis_index("s")

    pltpu.sync_copy(wr_hbm.at[sid], idx_v)
    copies = [
        pltpu.async_copy(scores_hbm.at[idx_v.at[pl.ds(i * 128, 128)]],
                         vals_v.at[i], sem)
        for i in range(NCHUNK)
    ]

    zidx_v[...] = jnp.zeros((LANES,), jnp.int32)

    @pl.when(sid == 0)
    def _():
        out_v[...] = jnp.zeros((LANES,), jnp.float32)
        pltpu.sync_copy(out_v, shared.at[0])

    for c in copies:
        c.wait()

    top = vals_v[0, pl.ds(0, LANES)][0]

    acc = jnp.zeros((LANES,), jnp.float32)
    for i in range(NCHUNK):
        for k in range(128 // LANES):
            x = vals_v[i, pl.ds(k * LANES, LANES)]
            acc = acc + jnp.maximum(x - top + MARGIN, 0.0)
    acc_v[0, pl.ds(0, LANES)] = acc

    plsc.subcore_barrier()
    pltpu.sync_copy(acc_v, shared.at[zidx_v.at[pl.ds(0, 1)]], add=True)
    plsc.subcore_barrier()

    @pl.when(sid == 0)
    def _():
        pltpu.sync_copy(shared.at[0], red_v)
        r = red_v[pl.ds(0, LANES)]
        s = (((r[0] + r[1]) + (r[2] + r[3])) + ((r[4] + r[5]) + (r[6] + r[7]))
             + (((r[8] + r[9]) + (r[10] + r[11]))
                + ((r[12] + r[13]) + (r[14] + r[15]))))
        y = (s - B * MARGIN) * (1.0 / (L - 1))
        out_v[...] = jnp.zeros((LANES,), jnp.float32) + y
        pltpu.sync_copy(out_v, out_hbm)


_mesh = plsc.VectorSubcoreMesh(core_axis_name="c", subcore_axis_name="s",
                               num_cores=1, num_subcores=NSUB)

_sc_call = pl.kernel(
    _sc_body,
    out_type=jax.ShapeDtypeStruct((LANES,), jnp.float32),
    mesh=_mesh,
    scratch_types=[
        pltpu.VMEM((L,), jnp.int32),
        pltpu.VMEM((NCHUNK, 128), jnp.float32),
        pltpu.VMEM((1, LANES), jnp.float32),
        pltpu.VMEM((LANES,), jnp.int32),
        pltpu.VMEM_SHARED((1, LANES), jnp.float32),
        pltpu.VMEM((LANES,), jnp.float32),
        pltpu.VMEM((LANES,), jnp.float32),
        pltpu.SemaphoreType.DMA,
    ],
    name="self_margin_loss_sc",
)


@jax.jit
def kernel(scores, nBestIndex, werRank):
    del nBestIndex
    return _sc_call(scores, werRank)[:1]


# speedup vs baseline: 1.0812x; 1.0089x over previous
import jax
import jax.numpy as jnp
from jax import lax
from jax.experimental import pallas as pl
from jax.experimental.pallas import tpu as pltpu
from jax.experimental.pallas import tpu_sc as plsc

B = 16
L = 2048
MARGIN = 0.1
LANES = 16
NSUB = 16
NCHUNK = L // 128


def _sc_body(scores_hbm, wr_hbm, out_hbm, idx_v, vals_v, acc_v, zidx_v,
             shared, red_v, out_v, sem):
    sid = lax.axis_index("s")

    pltpu.sync_copy(wr_hbm.at[sid], idx_v)
    copies = [
        pltpu.async_copy(scores_hbm.at[idx_v.at[pl.ds(i * 128, 128)]],
                         vals_v.at[i], sem)
        for i in range(NCHUNK)
    ]

    zidx_v[...] = jnp.zeros((LANES,), jnp.int32)

    @pl.when(sid == 0)
    def _():
        out_v[...] = jnp.zeros((LANES,), jnp.float32)
        pltpu.sync_copy(out_v, shared.at[0])

    acc = jnp.zeros((LANES,), jnp.float32)
    top = 0.0
    for i in range(NCHUNK):
        copies[i].wait()
        if i == 0:
            top = vals_v[0, pl.ds(0, LANES)][0]
        for k in range(128 // LANES):
            x = vals_v[i, pl.ds(k * LANES, LANES)]
            acc = acc + jnp.maximum(x - top + MARGIN, 0.0)
    acc_v[0, pl.ds(0, LANES)] = acc

    plsc.subcore_barrier()
    pltpu.sync_copy(acc_v, shared.at[zidx_v.at[pl.ds(0, 1)]], add=True)
    plsc.subcore_barrier()

    @pl.when(sid == 0)
    def _():
        pltpu.sync_copy(shared.at[0], red_v)
        r = red_v[pl.ds(0, LANES)]
        s = (((r[0] + r[1]) + (r[2] + r[3])) + ((r[4] + r[5]) + (r[6] + r[7]))
             + (((r[8] + r[9]) + (r[10] + r[11]))
                + ((r[12] + r[13]) + (r[14] + r[15]))))
        y = (s - B * MARGIN) * (1.0 / (L - 1))
        out_v[...] = jnp.zeros((LANES,), jnp.float32) + y
        pltpu.sync_copy(out_v, out_hbm)


_mesh = plsc.VectorSubcoreMesh(core_axis_name="c", subcore_axis_name="s",
                               num_cores=1, num_subcores=NSUB)

_sc_call = pl.kernel(
    _sc_body,
    out_type=jax.ShapeDtypeStruct((LANES,), jnp.float32),
    mesh=_mesh,
    scratch_types=[
        pltpu.VMEM((L,), jnp.int32),
        pltpu.VMEM((NCHUNK, 128), jnp.float32),
        pltpu.VMEM((1, LANES), jnp.float32),
        pltpu.VMEM((LANES,), jnp.int32),
        pltpu.VMEM_SHARED((1, LANES), jnp.float32),
        pltpu.VMEM((LANES,), jnp.float32),
        pltpu.VMEM((LANES,), jnp.float32),
        pltpu.SemaphoreType.DMA,
    ],
    name="self_margin_loss_sc",
)


@jax.jit
def kernel(scores, nBestIndex, werRank):
    del nBestIndex
    return _sc_call(scores, werRank)[:1]


# speedup vs baseline: 1.0853x; 1.0038x over previous
import jax
import jax.numpy as jnp
from jax import lax
from jax.experimental import pallas as pl
from jax.experimental.pallas import tpu as pltpu
from jax.experimental.pallas import tpu_sc as plsc

B = 16
L = 2048
MARGIN = 0.1
LANES = 16
NSUB = 16
NCHUNK = L // 128


def _sc_body(scores_hbm, wr_hbm, out_hbm, idx_v, vals_v, acc_v, zidx_v,
             shared, red_v, out_v, sem, isem):
    sid = lax.axis_index("s")

    idx_copies = [
        pltpu.async_copy(wr_hbm.at[sid, pl.ds(g * 512, 512)],
                         idx_v.at[pl.ds(g * 512, 512)], isem)
        for g in range(4)
    ]
    copies = []
    for g in range(4):
        idx_copies[g].wait()
        copies += [
            pltpu.async_copy(scores_hbm.at[idx_v.at[pl.ds(i * 128, 128)]],
                             vals_v.at[i], sem)
            for i in range(g * 4, g * 4 + 4)
        ]

    zidx_v[...] = jnp.zeros((LANES,), jnp.int32)

    @pl.when(sid == 0)
    def _():
        out_v[...] = jnp.zeros((LANES,), jnp.float32)
        pltpu.sync_copy(out_v, shared.at[0])

    acc = jnp.zeros((LANES,), jnp.float32)
    top = 0.0
    for i in range(NCHUNK):
        copies[i].wait()
        if i == 0:
            top = vals_v[0, pl.ds(0, LANES)][0]
        for k in range(128 // LANES):
            x = vals_v[i, pl.ds(k * LANES, LANES)]
            acc = acc + jnp.maximum(x - top + MARGIN, 0.0)
    acc_v[0, pl.ds(0, LANES)] = acc

    plsc.subcore_barrier()
    pltpu.sync_copy(acc_v, shared.at[zidx_v.at[pl.ds(0, 1)]], add=True)
    plsc.subcore_barrier()

    @pl.when(sid == 0)
    def _():
        pltpu.sync_copy(shared.at[0], red_v)
        r = red_v[pl.ds(0, LANES)]
        s = (((r[0] + r[1]) + (r[2] + r[3])) + ((r[4] + r[5]) + (r[6] + r[7]))
             + (((r[8] + r[9]) + (r[10] + r[11]))
                + ((r[12] + r[13]) + (r[14] + r[15]))))
        y = (s - B * MARGIN) * (1.0 / (L - 1))
        out_v[...] = jnp.zeros((LANES,), jnp.float32) + y
        pltpu.sync_copy(out_v, out_hbm)


_mesh = plsc.VectorSubcoreMesh(core_axis_name="c", subcore_axis_name="s",
                               num_cores=1, num_subcores=NSUB)

_sc_call = pl.kernel(
    _sc_body,
    out_type=jax.ShapeDtypeStruct((LANES,), jnp.float32),
    mesh=_mesh,
    scratch_types=[
        pltpu.VMEM((L,), jnp.int32),
        pltpu.VMEM((NCHUNK, 128), jnp.float32),
        pltpu.VMEM((1, LANES), jnp.float32),
        pltpu.VMEM((LANES,), jnp.int32),
        pltpu.VMEM_SHARED((1, LANES), jnp.float32),
        pltpu.VMEM((LANES,), jnp.float32),
        pltpu.VMEM((LANES,), jnp.float32),
        pltpu.SemaphoreType.DMA,
        pltpu.SemaphoreType.DMA,
    ],
    name="self_margin_loss_sc",
)


@jax.jit
def kernel(scores, nBestIndex, werRank):
    del nBestIndex
    return _sc_call(scores, werRank)[:1]
